# baseline (speedup 1.0000x reference)
import functools

import jax
import jax.numpy as jnp
from jax import lax
from jax.experimental import pallas as pl
from jax.experimental.pallas import tpu as pltpu
from jax.experimental.pallas import tpu_sc as plsc

_L = 16
_B, _T, _D = 16, 2048, 512
_NC, _NS = 2, 16
_NW = _NC * _NS
_HALF = _T // 2
_CHUNK = 64
_NCHUNK = _HALF // _CHUNK
_VPC = _CHUNK // _L
_NVEC = _T // _L


def _bcast_last(v):
    idx = jnp.full((_L, 1), _L - 1, jnp.int32)
    dn = lax.GatherDimensionNumbers(
        offset_dims=(), collapsed_slice_dims=(0,), start_index_map=(0,))
    return lax.gather(v, idx, dn, (1,),
                      mode=lax.GatherScatterMode.PROMISE_IN_BOUNDS)


def _body(x_hbm, dur_hbm, out_hbm, dur_v, ind_v, g_v,
          buf0, buf1, rs0, rs1, ws0, ws1, ds0):
    cid = lax.axis_index("c")
    sid = lax.axis_index("s")
    wid = sid * _NC + cid
    row = wid // 2
    half = wid % 2

    dur_cp = pltpu.async_copy(dur_hbm.at[row], dur_v, ds0)

    zeros = jnp.zeros((_L,), jnp.int32)

    def zero_body(k, carry):
        for u in range(8):
            ind_v[pl.ds((k * 8 + u) * _L, _L)] = zeros
        return carry

    lax.fori_loop(0, _NVEC // 8, zero_body, 0)
    dur_cp.wait()

    ones = jnp.ones((_L,), jnp.int32)

    def scat_body(k, carry):
        for u in range(4):
            d = dur_v[pl.ds((k * 4 + u) * _L, _L)]
            d = jnp.minimum(jnp.maximum(d, 0.0), 4096.0)
            r = (d + 0.5).astype(jnp.int32)
            incl = jnp.cumsum(r)
            excl = incl - r + carry
            plsc.addupdate_scatter(ind_v, (excl,), ones, mask=excl < _T)
            carry = carry + _bcast_last(incl)
        return carry

    lax.fori_loop(0, _NVEC // 4, scat_body, zeros)

    row_base = row * _T
    lo = half * (_HALF // _L)
    out_base = row_base + half * _HALF

    def presum_body(k, acc):
        return acc + ind_v[pl.ds(k * _L, _L)]

    acc = lax.fori_loop(0, lo, presum_body, zeros)
    carry = zeros + jnp.sum(acc)

    bufs, rsems, wsems = (buf0, buf1), (rs0, rs1), (ws0, ws1)
    gathers = [None, None]
    writes = [None, None]

    def out_slice(cc):
        return out_hbm.at[pl.ds(out_base + cc * _CHUNK, _CHUNK)]

    for cc in range(_NCHUNK):
        b = cc % 2
        if writes[b] is not None:
            writes[b].wait()
        for o in range(_VPC):
            k = lo + cc * _VPC + o
            ind = ind_v[pl.ds(k * _L, _L)]
            incl = jnp.cumsum(ind)
            g_v[cc, pl.ds(o * _L, _L)] = incl + carry - 1 + row_base
            carry = carry + _bcast_last(incl)
        gathers[b] = pltpu.async_copy(x_hbm.at[g_v.at[cc]], bufs[b], rsems[b])
        if cc >= 1:
            pb = 1 - b
            gathers[pb].wait()
            writes[pb] = pltpu.async_copy(bufs[pb], out_slice(cc - 1), wsems[pb])

    last = (_NCHUNK - 1) % 2
    gathers[last].wait()
    writes[last] = pltpu.async_copy(bufs[last], out_slice(_NCHUNK - 1), wsems[last])
    writes[0].wait()
    writes[1].wait()


_regulate = functools.partial(
    pl.kernel,
    out_type=jax.ShapeDtypeStruct((_B * _T, _D), jnp.float32),
    mesh=plsc.VectorSubcoreMesh(
        core_axis_name="c", subcore_axis_name="s",
        num_cores=_NC, num_subcores=_NS),
    compiler_params=pltpu.CompilerParams(needs_layout_passes=False),
    scratch_types=[
        pltpu.VMEM((_T,), jnp.float32),
        pltpu.VMEM((_T,), jnp.int32),
        pltpu.VMEM((_NCHUNK, _CHUNK), jnp.int32),
        pltpu.VMEM((_CHUNK, _D), jnp.float32),
        pltpu.VMEM((_CHUNK, _D), jnp.float32),
        pltpu.SemaphoreType.DMA,
        pltpu.SemaphoreType.DMA,
        pltpu.SemaphoreType.DMA,
        pltpu.SemaphoreType.DMA,
        pltpu.SemaphoreType.DMA,
    ],
)(_body)


def kernel(x, dur):
    out = _regulate(x.reshape(_B * _T, _D), dur)
    return out.reshape(_B, _T, _D)
